# baseline (speedup 1.0000x reference)
import jax
import jax.numpy as jnp
from jax import lax
from jax.experimental import pallas as pl
from jax.experimental.pallas import tpu as pltpu
from jax.experimental.pallas import tpu_sc as plsc

K_BINS = 1024
EMB = 64
TN = 512

_info = plsc.get_sparse_core_info()
_NW = _info.num_cores * _info.num_subcores


def _assign_body(z_ref, cb_ref, xl_ref, fit_ref):
    i = pl.program_id(0)
    x = z_ref[...]
    cb = cb_ref[...]
    xsq = jnp.sum(x * x, axis=1, keepdims=True)
    ksq = jnp.sum(cb * cb, axis=1)[None, :]
    xk2 = lax.dot_general(x, cb * (-2.0), (((1,), (1,)), ((), ())),
                          preferred_element_type=jnp.float32)
    dist = (xsq + xk2) + ksq
    mind = jnp.min(dist, axis=1)
    iota = lax.broadcasted_iota(jnp.int32, dist.shape, 1)
    idx = jnp.min(jnp.where(dist <= mind[:, None], iota, K_BINS), axis=1)
    xl_ref[...] = idx

    @pl.when(i == 0)
    def _init():
        fit_ref[...] = jnp.zeros((1, 1), jnp.float32)

    fit_ref[...] += jnp.sum(mind).reshape(1, 1)


def _gather_body(cb_hbm, xl_hbm, out_hbm, idx_v, rows_v, sem):
    bpw = out_hbm.shape[0] // _NW
    wid = lax.axis_index("s") * _info.num_cores + lax.axis_index("c")
    base = wid * bpw
    pltpu.sync_copy(xl_hbm.at[pl.ds(base, bpw)], idx_v)
    pltpu.async_copy(cb_hbm.at[idx_v], rows_v, sem).wait()
    pltpu.sync_copy(rows_v, out_hbm.at[pl.ds(base, bpw)])


@jax.jit
def kernel(z, codebook):
    B, T, D = z.shape
    N = B * T
    x = z.reshape(N, D)
    xl, fit_s = pl.pallas_call(
        _assign_body,
        grid=(N // TN,),
        in_specs=[
            pl.BlockSpec((TN, D), lambda i: (i, 0)),
            pl.BlockSpec((K_BINS, D), lambda i: (0, 0)),
        ],
        out_specs=[
            pl.BlockSpec((TN,), lambda i: (i,)),
            pl.BlockSpec((1, 1), lambda i: (0, 0)),
        ],
        out_shape=[
            jax.ShapeDtypeStruct((N,), jnp.int32),
            jax.ShapeDtypeStruct((1, 1), jnp.float32),
        ],
    )(x, codebook)

    bpw = N // _NW
    cb_pad = jnp.pad(codebook, ((0, 0), (0, 128 - D)))
    xq = pl.kernel(
        _gather_body,
        mesh=plsc.VectorSubcoreMesh(core_axis_name="c", subcore_axis_name="s"),
        out_type=jax.ShapeDtypeStruct((N, 128), jnp.float32),
        scratch_types=[
            pltpu.VMEM((bpw,), jnp.int32),
            pltpu.VMEM((bpw, 128), jnp.float32),
            pltpu.SemaphoreType.DMA,
        ],
    )(cb_pad, xl)

    fit = fit_s[0, 0] / N
    commit_loss = fit / D
    return xq[:, :D].reshape(B, T, D), commit_loss, fit, xl.reshape(B, T)


# speedup vs baseline: 1.0316x; 1.0316x over previous
import jax
import jax.numpy as jnp
from jax import lax
from jax.experimental import pallas as pl
from jax.experimental.pallas import tpu as pltpu
from jax.experimental.pallas import tpu_sc as plsc

K_BINS = 1024
EMB = 64
TN = 512

_info = plsc.get_sparse_core_info()
_NW = _info.num_cores * _info.num_subcores


def _assign_body(z_ref, cb_ref, xl_ref, fit_ref):
    i = pl.program_id(0)
    x = z_ref[...]
    cb = cb_ref[...]
    xsq = jnp.sum(x * x, axis=1, keepdims=True)
    ksq = jnp.sum(cb * cb, axis=1)[None, :]
    xk2 = lax.dot_general(x, cb * (-2.0), (((1,), (1,)), ((), ())),
                          preferred_element_type=jnp.float32)
    dist = (xsq + xk2) + ksq
    mind = jnp.min(dist, axis=1)
    idx = jnp.argmin(dist, axis=1).astype(jnp.int32)
    xl_ref[...] = idx

    @pl.when(i == 0)
    def _init():
        fit_ref[...] = jnp.zeros((1, 1), jnp.float32)

    fit_ref[...] += jnp.sum(mind).reshape(1, 1)


def _gather_body(cb_hbm, xl_hbm, out_hbm, idx_v, rows_v, sem):
    bpw = out_hbm.shape[0] // _NW
    wid = lax.axis_index("s") * _info.num_cores + lax.axis_index("c")
    base = wid * bpw
    pltpu.sync_copy(xl_hbm.at[pl.ds(base, bpw)], idx_v)
    pltpu.async_copy(cb_hbm.at[idx_v], rows_v, sem).wait()
    pltpu.sync_copy(rows_v, out_hbm.at[pl.ds(base, bpw)])


@jax.jit
def kernel(z, codebook):
    B, T, D = z.shape
    N = B * T
    x = z.reshape(N, D)
    xl, fit_s = pl.pallas_call(
        _assign_body,
        grid=(N // TN,),
        in_specs=[
            pl.BlockSpec((TN, D), lambda i: (i, 0)),
            pl.BlockSpec((K_BINS, D), lambda i: (0, 0)),
        ],
        out_specs=[
            pl.BlockSpec((TN,), lambda i: (i,)),
            pl.BlockSpec((1, 1), lambda i: (0, 0)),
        ],
        out_shape=[
            jax.ShapeDtypeStruct((N,), jnp.int32),
            jax.ShapeDtypeStruct((1, 1), jnp.float32),
        ],
    )(x, codebook)

    bpw = N // _NW
    cb_pad = jnp.pad(codebook, ((0, 0), (0, 128 - D)))
    xq = pl.kernel(
        _gather_body,
        mesh=plsc.VectorSubcoreMesh(core_axis_name="c", subcore_axis_name="s"),
        out_type=jax.ShapeDtypeStruct((N, 128), jnp.float32),
        scratch_types=[
            pltpu.VMEM((bpw,), jnp.int32),
            pltpu.VMEM((bpw, 128), jnp.float32),
            pltpu.SemaphoreType.DMA,
        ],
    )(cb_pad, xl)

    fit = fit_s[0, 0] / N
    commit_loss = fit / D
    return xq[:, :D].reshape(B, T, D), commit_loss, fit, xl.reshape(B, T)


# speedup vs baseline: 1.2150x; 1.1778x over previous
import jax
import jax.numpy as jnp
from jax import lax
from jax.experimental import pallas as pl
from jax.experimental.pallas import tpu as pltpu

K_BINS = 1024
EMB = 64
TN = 512


def _vq_body(z_ref, cb_ref, xq_ref, xl_ref, fit_ref):
    i = pl.program_id(0)
    x = z_ref[...]
    cb = cb_ref[...]
    xsq = jnp.sum(x * x, axis=1, keepdims=True)
    ksq = jnp.sum(cb * cb, axis=1)[None, :]
    xk2 = lax.dot_general(x, cb * (-2.0), (((1,), (1,)), ((), ())),
                          preferred_element_type=jnp.float32)
    dist = (xsq + xk2) + ksq
    mind = jnp.min(dist, axis=1)
    idx = jnp.argmin(dist, axis=1).astype(jnp.int32)
    onehot = (lax.broadcasted_iota(jnp.int32, dist.shape, 1)
              == idx[:, None]).astype(jnp.float32)
    xd = lax.dot_general(onehot, cb, (((1,), (0,)), ((), ())),
                         preferred_element_type=jnp.float32)
    xq_ref[...] = x + (xd - x)
    xl_ref[...] = idx

    @pl.when(i == 0)
    def _init():
        fit_ref[...] = jnp.zeros((1, 1), jnp.float32)

    fit_ref[...] += jnp.sum(mind).reshape(1, 1)


@jax.jit
def kernel(z, codebook):
    B, T, D = z.shape
    N = B * T
    x = z.reshape(N, D)
    xq, xl, fit_s = pl.pallas_call(
        _vq_body,
        grid=(N // TN,),
        in_specs=[
            pl.BlockSpec((TN, D), lambda i: (i, 0)),
            pl.BlockSpec((K_BINS, D), lambda i: (0, 0)),
        ],
        out_specs=[
            pl.BlockSpec((TN, D), lambda i: (i, 0)),
            pl.BlockSpec((TN,), lambda i: (i,)),
            pl.BlockSpec((1, 1), lambda i: (0, 0)),
        ],
        out_shape=[
            jax.ShapeDtypeStruct((N, D), jnp.float32),
            jax.ShapeDtypeStruct((N,), jnp.int32),
            jax.ShapeDtypeStruct((1, 1), jnp.float32),
        ],
    )(x, codebook)
    fit = fit_s[0, 0] / N
    commit_loss = fit / D
    return xq.reshape(B, T, D), commit_loss, fit, xl.reshape(B, T)


# speedup vs baseline: 1.3069x; 1.0757x over previous
import jax
import jax.numpy as jnp
from jax import lax
from jax.experimental import pallas as pl
from jax.experimental.pallas import tpu as pltpu

K_BINS = 1024
EMB = 64
TN = 512


def _vq_body(z_ref, cb_ref, xq_ref, xl_ref, fit_ref):
    i = pl.program_id(0)
    x = z_ref[...]
    cb = cb_ref[...]
    xsq = jnp.sum(x * x, axis=1, keepdims=True)
    ksq = jnp.sum(cb * cb, axis=1)[None, :]
    xk2 = lax.dot_general(x, cb * (-2.0), (((1,), (1,)), ((), ())),
                          preferred_element_type=jnp.float32)
    dist = (xsq + xk2) + ksq
    mind = jnp.min(dist, axis=1)
    iota = lax.broadcasted_iota(jnp.int32, dist.shape, 1)
    sel = jnp.where(dist <= mind[:, None], iota, K_BINS)
    idx = jnp.min(sel, axis=1)
    onehot = (iota == idx[:, None]).astype(jnp.float32)
    xd = lax.dot_general(onehot, cb, (((1,), (0,)), ((), ())),
                         preferred_element_type=jnp.float32)
    xq_ref[...] = x + (xd - x)
    xl_ref[...] = idx

    @pl.when(i == 0)
    def _init():
        fit_ref[...] = jnp.zeros((1, 1), jnp.float32)

    fit_ref[...] += jnp.sum(mind).reshape(1, 1)


@jax.jit
def kernel(z, codebook):
    B, T, D = z.shape
    N = B * T
    x = z.reshape(N, D)
    xq, xl, fit_s = pl.pallas_call(
        _vq_body,
        grid=(N // TN,),
        in_specs=[
            pl.BlockSpec((TN, D), lambda i: (i, 0)),
            pl.BlockSpec((K_BINS, D), lambda i: (0, 0)),
        ],
        out_specs=[
            pl.BlockSpec((TN, D), lambda i: (i, 0)),
            pl.BlockSpec((TN,), lambda i: (i,)),
            pl.BlockSpec((1, 1), lambda i: (0, 0)),
        ],
        out_shape=[
            jax.ShapeDtypeStruct((N, D), jnp.float32),
            jax.ShapeDtypeStruct((N,), jnp.int32),
            jax.ShapeDtypeStruct((1, 1), jnp.float32),
        ],
    )(x, codebook)
    fit = fit_s[0, 0] / N
    commit_loss = fit / D
    return xq.reshape(B, T, D), commit_loss, fit, xl.reshape(B, T)


# speedup vs baseline: 1.5577x; 1.1919x over previous
import jax
import jax.numpy as jnp
from jax import lax
from jax.experimental import pallas as pl
from jax.experimental.pallas import tpu as pltpu

K_BINS = 1024
EMB = 64
TN = 4096


def _vq_body(z_ref, cb_ref, xq_ref, xl_ref, fit_ref):
    i = pl.program_id(0)
    x = z_ref[...]
    cb = cb_ref[...]
    xsq = jnp.sum(x * x, axis=1, keepdims=True)
    ksq = jnp.sum(cb * cb, axis=1)[None, :]
    xk2 = lax.dot_general(x, cb * (-2.0), (((1,), (1,)), ((), ())),
                          preferred_element_type=jnp.float32)
    dist = (xsq + xk2) + ksq
    mind = jnp.min(dist, axis=1)
    iota = lax.broadcasted_iota(jnp.int32, dist.shape, 1)
    sel = jnp.where(dist <= mind[:, None], iota, K_BINS)
    idx = jnp.min(sel, axis=1)
    onehot = (iota == idx[:, None]).astype(jnp.float32)
    xd = lax.dot_general(onehot, cb, (((1,), (0,)), ((), ())),
                         preferred_element_type=jnp.float32)
    xq_ref[...] = x + (xd - x)
    xl_ref[...] = idx

    @pl.when(i == 0)
    def _init():
        fit_ref[...] = jnp.zeros((1, 1), jnp.float32)

    fit_ref[...] += jnp.sum(mind).reshape(1, 1)


@jax.jit
def kernel(z, codebook):
    B, T, D = z.shape
    N = B * T
    x = z.reshape(N, D)
    xq, xl, fit_s = pl.pallas_call(
        _vq_body,
        grid=(N // TN,),
        in_specs=[
            pl.BlockSpec((TN, D), lambda i: (i, 0)),
            pl.BlockSpec((K_BINS, D), lambda i: (0, 0)),
        ],
        out_specs=[
            pl.BlockSpec((TN, D), lambda i: (i, 0)),
            pl.BlockSpec((TN,), lambda i: (i,)),
            pl.BlockSpec((1, 1), lambda i: (0, 0)),
        ],
        out_shape=[
            jax.ShapeDtypeStruct((N, D), jnp.float32),
            jax.ShapeDtypeStruct((N,), jnp.int32),
            jax.ShapeDtypeStruct((1, 1), jnp.float32),
        ],
    )(x, codebook)
    fit = fit_s[0, 0] / N
    commit_loss = fit / D
    return xq.reshape(B, T, D), commit_loss, fit, xl.reshape(B, T)
